# Initial kernel scaffold; baseline (speedup 1.0000x reference)
import functools

import jax
import jax.numpy as jnp
from jax import lax
from jax.experimental import pallas as pl
from jax.experimental.pallas import tpu as pltpu
from jax.experimental.pallas import tpu_sc as plsc

B, L, C = 1024, 20, 128
NW = 32
BPW = B // NW
IPW = BPW * L
ICH = IPW // 128


def _sc_pool_body(cat_i, gen_i, sty_i, cat_t, gen_t, sty_t, out_hbm,
                  idx_v, rows_v, out_v, sem):
    nc = plsc.get_sparse_core_info().num_cores
    wid = lax.axis_index("s") * nc + lax.axis_index("c")

    for f, (idx_hbm, table) in enumerate(
            ((cat_i, cat_t), (gen_i, gen_t), (sty_i, sty_t))):
        pltpu.sync_copy(idx_hbm.at[pl.ds(wid * ICH, ICH)], idx_v)
        cps = [
            pltpu.async_copy(table.at[idx_v.at[j]],
                             rows_v.at[pl.ds(j * 128, 128)], sem)
            for j in range(ICH)
        ]
        for cp in cps:
            cp.wait()

        off = f * C

        def body(e, carry):
            base = e * L
            for c in range(C // 16):
                sl = pl.ds(16 * c, 16)
                acc = rows_v[base, sl]
                for l in range(1, L):
                    acc = acc + rows_v[base + l, sl]
                out_v[e, pl.ds(off + 16 * c, 16)] = acc * (1.0 / L)
            return carry

        lax.fori_loop(0, BPW, body, 0)

    pltpu.sync_copy(out_v, out_hbm.at[pl.ds(wid * BPW, BPW)])


def _sc_pool(cat_i, gen_i, sty_i, cat_t, gen_t, sty_t):
    mesh = plsc.VectorSubcoreMesh(core_axis_name="c", subcore_axis_name="s")
    return pl.kernel(
        _sc_pool_body,
        out_type=jax.ShapeDtypeStruct((B, 3 * C), jnp.float32),
        mesh=mesh,
        scratch_types=[
            pltpu.VMEM((ICH, 128), jnp.int32),
            pltpu.VMEM((IPW, C), jnp.float32),
            pltpu.VMEM((BPW, 3 * C), jnp.float32),
            pltpu.SemaphoreType.DMA,
        ],
    )(cat_i, gen_i, sty_i, cat_t, gen_t, sty_t)


def _silu(x):
    return x * jax.nn.sigmoid(x)


def _mm(x, w):
    return jnp.dot(x, w, preferred_element_type=jnp.float32,
                   precision=lax.Precision.HIGHEST)


def _tc_dense_body(emb_ref, eps_ref, cW1, cb1, cW2, cb2, gW1, gb1, gW2, gb2,
                   sW1, sb1, sW2, sb2, muW1, mub1, muW2, mub2, vW1, vb1,
                   vW2, vb2, out_ref):
    emb = emb_ref[...]
    ec = _silu(_mm(_silu(_mm(emb[:, 0 * C:1 * C], cW1[...]) + cb1[...]),
                   cW2[...]) + cb2[...])
    eg = _silu(_mm(_silu(_mm(emb[:, 1 * C:2 * C], gW1[...]) + gb1[...]),
                   gW2[...]) + gb2[...])
    es = _silu(_mm(_silu(_mm(emb[:, 2 * C:3 * C], sW1[...]) + sb1[...]),
                   sW2[...]) + sb2[...])
    cat = jnp.concatenate([ec, eg, es], axis=1)
    mu = _mm(jax.nn.relu(_mm(cat, muW1[...]) + mub1[...]), muW2[...]) \
        + mub2[...]
    var = _mm(jax.nn.relu(_mm(cat, vW1[...]) + vb1[...]), vW2[...]) \
        + vb2[...]
    out_ref[...] = mu + jnp.exp(0.5 * var) * eps_ref[...]


def _tc_dense(emb, eps, weights):
    blk = 256
    grid = B // blk
    row_spec = lambda w: pl.BlockSpec((blk, w), lambda i: (i, 0))
    full = lambda a: pl.BlockSpec(a.shape, lambda i: (0,) * a.ndim)
    return pl.pallas_call(
        _tc_dense_body,
        grid=(grid,),
        in_specs=[row_spec(3 * C), row_spec(C)] + [full(w) for w in weights],
        out_specs=row_spec(C),
        out_shape=jax.ShapeDtypeStruct((B, C), jnp.float32),
    )(emb, eps, *weights)


def kernel(category, genre, style, cat_table, gen_table, sty_table,
           cW1, cb1, cW2, cb2, gW1, gb1, gW2, gb2, sW1, sb1, sW2, sb2,
           muW1, mub1, muW2, mub2, vW1, vb1, vW2, vb2, eps):
    cat_i = category.reshape(B * L // 128, 128)
    gen_i = genre.reshape(B * L // 128, 128)
    sty_i = style.reshape(B * L // 128, 128)
    emb = _sc_pool(cat_i, gen_i, sty_i, cat_table, gen_table, sty_table)
    weights = [cW1, cb1.reshape(1, -1), cW2, cb2.reshape(1, -1),
               gW1, gb1.reshape(1, -1), gW2, gb2.reshape(1, -1),
               sW1, sb1.reshape(1, -1), sW2, sb2.reshape(1, -1),
               muW1, mub1.reshape(1, -1), muW2, mub2.reshape(1, -1),
               vW1, vb1.reshape(1, -1), vW2, vb2.reshape(1, -1)]
    return _tc_dense(emb, eps, weights)


# speedup vs baseline: 1.7915x; 1.7915x over previous
import functools

import jax
import jax.numpy as jnp
from jax import lax
from jax.experimental import pallas as pl
from jax.experimental.pallas import tpu as pltpu
from jax.experimental.pallas import tpu_sc as plsc

B, L, C = 1024, 20, 128
NW = 32
BPW = B // NW
IPW = BPW * L
ICH = IPW // 128


def _sc_pool_body(cat_i, gen_i, sty_i, cat_t, gen_t, sty_t, out_hbm,
                  idx_v, rows_v, out_v, sem):
    nc = plsc.get_sparse_core_info().num_cores
    wid = lax.axis_index("s") * nc + lax.axis_index("c")

    for f, (idx_hbm, table) in enumerate(
            ((cat_i, cat_t), (gen_i, gen_t), (sty_i, sty_t))):
        pltpu.sync_copy(idx_hbm.at[pl.ds(wid * IPW, IPW)], idx_v)
        cps = [
            pltpu.async_copy(table.at[idx_v.at[pl.ds(j * 128, 128)]],
                             rows_v.at[pl.ds(j * 128, 128)], sem)
            for j in range(ICH)
        ]
        for cp in cps:
            cp.wait()

        off = f * C

        def body(e, carry):
            base = e * L
            for c in range(C // 16):
                sl = pl.ds(16 * c, 16)
                acc = rows_v[base, sl]
                for l in range(1, L):
                    acc = acc + rows_v[base + l, sl]
                out_v[e, pl.ds(off + 16 * c, 16)] = acc * (1.0 / L)
            return carry

        lax.fori_loop(0, BPW, body, 0)

    pltpu.sync_copy(out_v, out_hbm.at[pl.ds(wid * BPW, BPW)])


def _sc_pool(cat_i, gen_i, sty_i, cat_t, gen_t, sty_t):
    mesh = plsc.VectorSubcoreMesh(core_axis_name="c", subcore_axis_name="s")
    return pl.kernel(
        _sc_pool_body,
        out_type=jax.ShapeDtypeStruct((B, 3 * C), jnp.float32),
        mesh=mesh,
        scratch_types=[
            pltpu.VMEM((IPW,), jnp.int32),
            pltpu.VMEM((IPW, C), jnp.float32),
            pltpu.VMEM((BPW, 3 * C), jnp.float32),
            pltpu.SemaphoreType.DMA,
        ],
    )(cat_i, gen_i, sty_i, cat_t, gen_t, sty_t)


def _silu(x):
    return x * jax.nn.sigmoid(x)


def _mm(x, w):
    return jnp.dot(x, w, preferred_element_type=jnp.float32,
                   precision=lax.Precision.HIGHEST)


def _tc_dense_body(emb_ref, eps_ref, cW1, cb1, cW2, cb2, gW1, gb1, gW2, gb2,
                   sW1, sb1, sW2, sb2, muW1, mub1, muW2, mub2, vW1, vb1,
                   vW2, vb2, out_ref):
    emb = emb_ref[...]
    ec = _silu(_mm(_silu(_mm(emb[:, 0 * C:1 * C], cW1[...]) + cb1[...]),
                   cW2[...]) + cb2[...])
    eg = _silu(_mm(_silu(_mm(emb[:, 1 * C:2 * C], gW1[...]) + gb1[...]),
                   gW2[...]) + gb2[...])
    es = _silu(_mm(_silu(_mm(emb[:, 2 * C:3 * C], sW1[...]) + sb1[...]),
                   sW2[...]) + sb2[...])
    cat = jnp.concatenate([ec, eg, es], axis=1)
    mu = _mm(jax.nn.relu(_mm(cat, muW1[...]) + mub1[...]), muW2[...]) \
        + mub2[...]
    var = _mm(jax.nn.relu(_mm(cat, vW1[...]) + vb1[...]), vW2[...]) \
        + vb2[...]
    out_ref[...] = mu + jnp.exp(0.5 * var) * eps_ref[...]


def _tc_dense(emb, eps, weights):
    blk = 256
    grid = B // blk
    row_spec = lambda w: pl.BlockSpec((blk, w), lambda i: (i, 0))
    full = lambda a: pl.BlockSpec(a.shape, lambda i: (0,) * a.ndim)
    return pl.pallas_call(
        _tc_dense_body,
        grid=(grid,),
        in_specs=[row_spec(3 * C), row_spec(C)] + [full(w) for w in weights],
        out_specs=row_spec(C),
        out_shape=jax.ShapeDtypeStruct((B, C), jnp.float32),
    )(emb, eps, *weights)


def kernel(category, genre, style, cat_table, gen_table, sty_table,
           cW1, cb1, cW2, cb2, gW1, gb1, gW2, gb2, sW1, sb1, sW2, sb2,
           muW1, mub1, muW2, mub2, vW1, vb1, vW2, vb2, eps):
    cat_i = category.reshape(B * L)
    gen_i = genre.reshape(B * L)
    sty_i = style.reshape(B * L)
    emb = _sc_pool(cat_i, gen_i, sty_i, cat_table, gen_table, sty_table)
    weights = [cW1, cb1.reshape(1, -1), cW2, cb2.reshape(1, -1),
               gW1, gb1.reshape(1, -1), gW2, gb2.reshape(1, -1),
               sW1, sb1.reshape(1, -1), sW2, sb2.reshape(1, -1),
               muW1, mub1.reshape(1, -1), muW2, mub2.reshape(1, -1),
               vW1, vb1.reshape(1, -1), vW2, vb2.reshape(1, -1)]
    return _tc_dense(emb, eps, weights)


# speedup vs baseline: 3.1810x; 1.7756x over previous
import functools

import jax
import jax.numpy as jnp
from jax import lax
from jax.experimental import pallas as pl
from jax.experimental.pallas import tpu as pltpu
from jax.experimental.pallas import tpu_sc as plsc

B, L, C = 1024, 20, 128
NW = 32
BPW = B // NW
IPW = BPW * L
ICH = IPW // 128


def _sc_pool_body(cat_i, gen_i, sty_i, cat_t, gen_t, sty_t, out_hbm,
                  idx_v, rows_v, out_v, sem):
    nc = plsc.get_sparse_core_info().num_cores
    wid = lax.axis_index("s") * nc + lax.axis_index("c")
    nrows = (16, 64, 128)

    for f, (idx_hbm, table) in enumerate(
            ((cat_i, cat_t), (gen_i, gen_t), (sty_i, sty_t))):
        pltpu.sync_copy(idx_hbm.at[pl.ds(wid * IPW, IPW)], idx_v)
        roff = (wid * nrows[f]).astype(jnp.int32)
        for k in range(IPW // 16):
            sl16 = pl.ds(k * 16, 16)
            idx_v[sl16] = idx_v[sl16] + roff
        cps = [
            pltpu.async_copy(table.at[idx_v.at[pl.ds(j * 128, 128)]],
                             rows_v.at[pl.ds(j * 128, 128)], sem)
            for j in range(ICH)
        ]
        for cp in cps:
            cp.wait()

        off = f * C

        def body(e, carry):
            base = e * L
            for c in range(C // 16):
                sl = pl.ds(16 * c, 16)
                acc = rows_v[base, sl]
                for l in range(1, L):
                    acc = acc + rows_v[base + l, sl]
                out_v[e, pl.ds(off + 16 * c, 16)] = acc * (1.0 / L)
            return carry

        lax.fori_loop(0, BPW, body, 0)

    pltpu.sync_copy(out_v, out_hbm.at[pl.ds(wid * BPW, BPW)])


def _sc_pool(cat_i, gen_i, sty_i, cat_t, gen_t, sty_t):
    mesh = plsc.VectorSubcoreMesh(core_axis_name="c", subcore_axis_name="s")
    return pl.kernel(
        _sc_pool_body,
        out_type=jax.ShapeDtypeStruct((B, 3 * C), jnp.float32),
        mesh=mesh,
        scratch_types=[
            pltpu.VMEM((IPW,), jnp.int32),
            pltpu.VMEM((IPW, C), jnp.float32),
            pltpu.VMEM((BPW, 3 * C), jnp.float32),
            pltpu.SemaphoreType.DMA,
        ],
    )(cat_i, gen_i, sty_i, cat_t, gen_t, sty_t)


def _silu(x):
    return x * jax.nn.sigmoid(x)


def _mm(x, w):
    return jnp.dot(x, w, preferred_element_type=jnp.float32,
                   precision=lax.Precision.HIGHEST)


def _tc_dense_body(emb_ref, eps_ref, cW1, cb1, cW2, cb2, gW1, gb1, gW2, gb2,
                   sW1, sb1, sW2, sb2, muW1, mub1, muW2, mub2, vW1, vb1,
                   vW2, vb2, out_ref):
    emb = emb_ref[...]
    ec = _silu(_mm(_silu(_mm(emb[:, 0 * C:1 * C], cW1[...]) + cb1[...]),
                   cW2[...]) + cb2[...])
    eg = _silu(_mm(_silu(_mm(emb[:, 1 * C:2 * C], gW1[...]) + gb1[...]),
                   gW2[...]) + gb2[...])
    es = _silu(_mm(_silu(_mm(emb[:, 2 * C:3 * C], sW1[...]) + sb1[...]),
                   sW2[...]) + sb2[...])
    cat = jnp.concatenate([ec, eg, es], axis=1)
    mu = _mm(jax.nn.relu(_mm(cat, muW1[...]) + mub1[...]), muW2[...]) \
        + mub2[...]
    var = _mm(jax.nn.relu(_mm(cat, vW1[...]) + vb1[...]), vW2[...]) \
        + vb2[...]
    out_ref[...] = mu + jnp.exp(0.5 * var) * eps_ref[...]


def _tc_dense(emb, eps, weights):
    blk = 256
    grid = B // blk
    row_spec = lambda w: pl.BlockSpec((blk, w), lambda i: (i, 0))
    full = lambda a: pl.BlockSpec(a.shape, lambda i: (0,) * a.ndim)
    return pl.pallas_call(
        _tc_dense_body,
        grid=(grid,),
        in_specs=[row_spec(3 * C), row_spec(C)] + [full(w) for w in weights],
        out_specs=row_spec(C),
        out_shape=jax.ShapeDtypeStruct((B, C), jnp.float32),
    )(emb, eps, *weights)


def kernel(category, genre, style, cat_table, gen_table, sty_table,
           cW1, cb1, cW2, cb2, gW1, gb1, gW2, gb2, sW1, sb1, sW2, sb2,
           muW1, mub1, muW2, mub2, vW1, vb1, vW2, vb2, eps):
    cat_i = category.reshape(B * L)
    gen_i = genre.reshape(B * L)
    sty_i = style.reshape(B * L)
    emb = _sc_pool(cat_i, gen_i, sty_i, jnp.tile(cat_table, (NW, 1)),
                   jnp.tile(gen_table, (NW, 1)), jnp.tile(sty_table, (NW, 1)))
    weights = [cW1, cb1.reshape(1, -1), cW2, cb2.reshape(1, -1),
               gW1, gb1.reshape(1, -1), gW2, gb2.reshape(1, -1),
               sW1, sb1.reshape(1, -1), sW2, sb2.reshape(1, -1),
               muW1, mub1.reshape(1, -1), muW2, mub2.reshape(1, -1),
               vW1, vb1.reshape(1, -1), vW2, vb2.reshape(1, -1)]
    return _tc_dense(emb, eps, weights)


# speedup vs baseline: 3.5535x; 1.1171x over previous
import jax
import jax.numpy as jnp
from jax import lax
from jax.experimental import pallas as pl
from jax.experimental.pallas import tpu as pltpu
from jax.experimental.pallas import tpu_sc as plsc

B, L, C = 1024, 20, 128
NW = 32
BPW = B // NW
IPW = BPW * L
NV = (16, 64, 128)
TBASE = (0, NW * 16, NW * (16 + 64))
HPW = BPW // 2
RPP = HPW * L


def _sc_pool_body(idx_hbm, table, out_hbm, idx_v, rows0, rows1, out_v,
                  sem0, sem1):
    nc = plsc.get_sparse_core_info().num_cores
    wid = lax.axis_index("s") * nc + lax.axis_index("c")

    pltpu.sync_copy(idx_hbm.at[pl.ds(wid * 3 * IPW, 3 * IPW)], idx_v)
    roffs = [(TBASE[f] + wid * NV[f]).astype(jnp.int32) for f in range(3)]
    for k in range(3 * IPW // 16):
        sl16 = pl.ds(k * 16, 16)
        idx_v[sl16] = idx_v[sl16] + roffs[k // (IPW // 16)]

    rows = (rows0, rows1)
    sems = (sem0, sem1)

    def fire(p):
        f, h = divmod(p, 2)
        base = f * IPW + h * RPP
        buf, sem = rows[p % 2], sems[p % 2]
        return [
            pltpu.async_copy(table.at[idx_v.at[pl.ds(base + o, n)]],
                             buf.at[pl.ds(o, n)], sem)
            for o, n in ((0, 128), (128, 128), (256, 64))
        ]

    pending = fire(0)
    for p in range(6):
        nxt = fire(p + 1) if p < 5 else []
        for cp in pending:
            cp.wait()
        pending = nxt

        f, h = divmod(p, 2)
        buf = rows[p % 2]
        off = f * C

        def body(e, carry, buf=buf, off=off, h=h):
            base = e * L
            for c in range(C // 16):
                sl = pl.ds(16 * c, 16)
                acc = buf[base, sl]
                for l in range(1, L):
                    acc = acc + buf[base + l, sl]
                out_v[h * HPW + e, pl.ds(off + 16 * c, 16)] = acc * (1.0 / L)
            return carry

        lax.fori_loop(0, HPW, body, 0)

    pltpu.sync_copy(out_v, out_hbm.at[pl.ds(wid * BPW, BPW)])


def _sc_pool(idx_all, table_all):
    mesh = plsc.VectorSubcoreMesh(core_axis_name="c", subcore_axis_name="s")
    return pl.kernel(
        _sc_pool_body,
        out_type=jax.ShapeDtypeStruct((B, 3 * C), jnp.float32),
        mesh=mesh,
        scratch_types=[
            pltpu.VMEM((3 * IPW,), jnp.int32),
            pltpu.VMEM((RPP, C), jnp.float32),
            pltpu.VMEM((RPP, C), jnp.float32),
            pltpu.VMEM((BPW, 3 * C), jnp.float32),
            pltpu.SemaphoreType.DMA,
            pltpu.SemaphoreType.DMA,
        ],
    )(idx_all, table_all)


def _silu(x):
    return x * jax.nn.sigmoid(x)


def _mm(x, w):
    return jnp.dot(x, w, preferred_element_type=jnp.float32,
                   precision=lax.Precision.HIGHEST)


def _tc_dense_body(emb_ref, eps_ref, cW1, cb1, cW2, cb2, gW1, gb1, gW2, gb2,
                   sW1, sb1, sW2, sb2, muW1, mub1, muW2, mub2, vW1, vb1,
                   vW2, vb2, out_ref):
    emb = emb_ref[...]
    ec = _silu(_mm(_silu(_mm(emb[:, 0 * C:1 * C], cW1[...]) + cb1[...]),
                   cW2[...]) + cb2[...])
    eg = _silu(_mm(_silu(_mm(emb[:, 1 * C:2 * C], gW1[...]) + gb1[...]),
                   gW2[...]) + gb2[...])
    es = _silu(_mm(_silu(_mm(emb[:, 2 * C:3 * C], sW1[...]) + sb1[...]),
                   sW2[...]) + sb2[...])
    cat = jnp.concatenate([ec, eg, es], axis=1)
    mu = _mm(jax.nn.relu(_mm(cat, muW1[...]) + mub1[...]), muW2[...]) \
        + mub2[...]
    var = _mm(jax.nn.relu(_mm(cat, vW1[...]) + vb1[...]), vW2[...]) \
        + vb2[...]
    out_ref[...] = mu + jnp.exp(0.5 * var) * eps_ref[...]


def _tc_dense(emb, eps, weights):
    blk = 256
    grid = B // blk
    row_spec = lambda w: pl.BlockSpec((blk, w), lambda i: (i, 0))
    full = lambda a: pl.BlockSpec(a.shape, lambda i: (0,) * a.ndim)
    return pl.pallas_call(
        _tc_dense_body,
        grid=(grid,),
        in_specs=[row_spec(3 * C), row_spec(C)] + [full(w) for w in weights],
        out_specs=row_spec(C),
        out_shape=jax.ShapeDtypeStruct((B, C), jnp.float32),
    )(emb, eps, *weights)


def kernel(category, genre, style, cat_table, gen_table, sty_table,
           cW1, cb1, cW2, cb2, gW1, gb1, gW2, gb2, sW1, sb1, sW2, sb2,
           muW1, mub1, muW2, mub2, vW1, vb1, vW2, vb2, eps):
    idx_all = jnp.stack([category.reshape(NW, IPW),
                         genre.reshape(NW, IPW),
                         style.reshape(NW, IPW)], axis=1).reshape(-1)
    table_all = jnp.concatenate([jnp.tile(cat_table, (NW, 1)),
                                 jnp.tile(gen_table, (NW, 1)),
                                 jnp.tile(sty_table, (NW, 1))], axis=0)
    emb = _sc_pool(idx_all, table_all)
    weights = [cW1, cb1.reshape(1, -1), cW2, cb2.reshape(1, -1),
               gW1, gb1.reshape(1, -1), gW2, gb2.reshape(1, -1),
               sW1, sb1.reshape(1, -1), sW2, sb2.reshape(1, -1),
               muW1, mub1.reshape(1, -1), muW2, mub2.reshape(1, -1),
               vW1, vb1.reshape(1, -1), vW2, vb2.reshape(1, -1)]
    return _tc_dense(emb, eps, weights)
